# Initial kernel scaffold; baseline (speedup 1.0000x reference)
import jax
import jax.numpy as jnp
from jax.experimental import pallas as pl


def kernel(x_user, batch_user, x_news, batch_news, news_embeddings, W1, b1, W2, b2, W3, b3):
    raise NotImplementedError("write your pallas kernel here")


# speedup vs baseline: 4.4726x; 4.4726x over previous
import functools

import jax
import jax.numpy as jnp
from jax import lax
from jax.experimental import pallas as pl
from jax.experimental.pallas import tpu as pltpu
from jax.experimental.pallas import tpu_sc as plsc

B = 1024
N = 100000
D = 128

NUM_CORES = 2
NUM_SUBCORES = 16
CHUNK = 128
ROWS_PER_TILE = 6272
FULL_TILES_ROWS = 15 * ROWS_PER_TILE
LAST_BASE = FULL_TILES_ROWS
LAST_FULL_CHUNKS = (N - LAST_BASE) // CHUNK
TAIL = N - LAST_BASE - LAST_FULL_CHUNKS * CHUNK
SEGS_PER_TILE = B // NUM_SUBCORES
CW = 128


def _sc_body(xu, bu, xn, bn, sums_out, cnts_out,
             xbuf, idxbuf, idxbuf_t, onesbuf, zbuf, zbuf16, acc, cnt):
    c = lax.axis_index("c")
    s = lax.axis_index("s")

    def z128(i, carry):
        for j in range(D // 16):
            zbuf[i, pl.ds(16 * j, 16)] = jnp.zeros((16,), jnp.float32)
        return carry

    lax.fori_loop(0, SEGS_PER_TILE, z128, None)

    def z16(i, carry):
        zbuf16[i, pl.ds(0, 16)] = jnp.zeros((16,), jnp.float32)
        return carry

    lax.fori_loop(0, SEGS_PER_TILE, z16, None)

    def o16(i, carry):
        onesbuf[i, pl.ds(0, 16)] = jnp.ones((16,), jnp.float32)
        return carry

    lax.fori_loop(0, CHUNK, o16, None)

    seg0 = s * SEGS_PER_TILE
    pltpu.sync_copy(zbuf, acc.at[pl.ds(seg0, SEGS_PER_TILE)])
    pltpu.sync_copy(zbuf16, cnt.at[pl.ds(seg0, SEGS_PER_TILE)])
    plsc.subcore_barrier()

    def process(x_hbm, idx_hbm):
        base = s * ROWS_PER_TILE

        def chunk_step(j, carry):
            row0 = base + j * CHUNK
            pltpu.sync_copy(idx_hbm.at[pl.ds(row0, CHUNK)], idxbuf)
            pltpu.sync_copy(x_hbm.at[pl.ds(row0, CHUNK), :], xbuf)
            pltpu.sync_copy(xbuf, acc.at[idxbuf], add=True)
            pltpu.sync_copy(onesbuf, cnt.at[idxbuf], add=True)
            return carry

        @pl.when(s < NUM_SUBCORES - 1)
        def _():
            lax.fori_loop(0, ROWS_PER_TILE // CHUNK, chunk_step, None)

        @pl.when(s == NUM_SUBCORES - 1)
        def _():
            lax.fori_loop(0, LAST_FULL_CHUNKS, chunk_step, None)
            row0 = LAST_BASE + LAST_FULL_CHUNKS * CHUNK
            pltpu.sync_copy(idx_hbm.at[pl.ds(row0, TAIL)], idxbuf_t)
            pltpu.sync_copy(x_hbm.at[pl.ds(row0, TAIL), :],
                            xbuf.at[pl.ds(0, TAIL), :])
            pltpu.sync_copy(xbuf.at[pl.ds(0, TAIL), :], acc.at[idxbuf_t],
                            add=True)
            pltpu.sync_copy(onesbuf.at[pl.ds(0, TAIL), :], cnt.at[idxbuf_t],
                            add=True)

    @pl.when(c == 0)
    def _():
        process(xu, bu)

    @pl.when(c == 1)
    def _():
        process(xn, bn)

    plsc.subcore_barrier()
    out0 = c * B + seg0
    pltpu.sync_copy(acc.at[pl.ds(seg0, SEGS_PER_TILE)],
                    sums_out.at[pl.ds(out0, SEGS_PER_TILE)])
    pltpu.sync_copy(cnt.at[pl.ds(seg0, SEGS_PER_TILE)],
                    cnts_out.at[pl.ds(out0, SEGS_PER_TILE)])


def _segment_sums(x_user, batch_user, x_news, batch_news):
    mesh = plsc.VectorSubcoreMesh(core_axis_name="c", subcore_axis_name="s")
    fn = pl.kernel(
        _sc_body,
        mesh=mesh,
        out_type=[
            jax.ShapeDtypeStruct((NUM_CORES * B, D), jnp.float32),
            jax.ShapeDtypeStruct((NUM_CORES * B, CW), jnp.float32),
        ],
        scratch_types=[
            pltpu.VMEM((CHUNK, D), jnp.float32),
            pltpu.VMEM((CHUNK,), jnp.int32),
            pltpu.VMEM((TAIL,), jnp.int32),
            pltpu.VMEM((CHUNK, CW), jnp.float32),
            pltpu.VMEM((SEGS_PER_TILE, D), jnp.float32),
            pltpu.VMEM((SEGS_PER_TILE, CW), jnp.float32),
            pltpu.VMEM_SHARED((B, D), jnp.float32),
            pltpu.VMEM_SHARED((B, CW), jnp.float32),
        ],
    )
    return fn(x_user, batch_user, x_news, batch_news)


def _mlp_body(sums_ref, cnts_ref, ne_ref, w1_ref, b1_ref, w2_ref, b2_ref,
              w3_ref, b3_ref, out_ref):
    su = sums_ref[0:B, :]
    sn = sums_ref[B:2 * B, :]
    cu = cnts_ref[0:B, 0:1]
    cn = cnts_ref[B:2 * B, 0:1]
    pu = su / jnp.maximum(cu, 1.0)
    pn = sn / jnp.maximum(cn, 1.0)
    hp = jax.lax.Precision.HIGHEST
    h = jnp.dot(pu, w1_ref[0:D, :], precision=hp)
    h = h + jnp.dot(pn, w1_ref[D:2 * D, :], precision=hp)
    h = jnp.maximum(h + b1_ref[0:1, :], 0.0)
    h = jnp.maximum(jnp.dot(h, w2_ref[...], precision=hp) + b2_ref[0:1, :], 0.0)
    out_ref[...] = (jnp.dot(h, w3_ref[...], precision=hp) + b3_ref[0:1, :]
                    + ne_ref[...])


def kernel(x_user, batch_user, x_news, batch_news, news_embeddings,
           W1, b1, W2, b2, W3, b3):
    bu = batch_user.astype(jnp.int32)
    bn = batch_news.astype(jnp.int32)
    sums, cnts = _segment_sums(x_user, bu, x_news, bn)
    return pl.pallas_call(
        _mlp_body,
        out_shape=jax.ShapeDtypeStruct((B, D), jnp.float32),
    )(sums, cnts, news_embeddings,
      W1, b1.reshape(1, D), W2, b2.reshape(1, D), W3, b3.reshape(1, D))


# speedup vs baseline: 5.1896x; 1.1603x over previous
import jax
import jax.numpy as jnp
from jax import lax
from jax.experimental import pallas as pl
from jax.experimental.pallas import tpu as pltpu
from jax.experimental.pallas import tpu_sc as plsc

B = 1024
N = 100000
D = 128

NUM_CORES = 2
NUM_SUBCORES = 16
L = 16
CHUNK = 128
ROWS_PER_TILE = 6272
FULL_CHUNKS = ROWS_PER_TILE // CHUNK
LAST_BASE = 15 * ROWS_PER_TILE
LAST_FULL_CHUNKS = (N - LAST_BASE) // CHUNK
TAIL = N - LAST_BASE - LAST_FULL_CHUNKS * CHUNK
SEGS_PER_TILE = B // NUM_SUBCORES


def _zero_2d(ref, rows, cols):
    def body(i, carry):
        for j in range(cols // L):
            ref[i, pl.ds(L * j, L)] = jnp.zeros((L,), jnp.float32)
        return carry

    lax.fori_loop(0, rows, body, None)


def _count_chunk(cnt2, idx_ref, rows):
    lane_off = lax.iota(jnp.int32, L) * B
    ones = jnp.ones((L,), jnp.float32)
    for g in range(rows // L):
        idx16 = idx_ref[pl.ds(L * g, L)]
        plsc.addupdate_scatter(cnt2, [lane_off + idx16], ones)


def _sc_body(xu, bu, xn, bn, means_out,
             xbuf, idxbuf, idxbuf_t, zbuf, cnt2, lcnt, cbuf, csum, dbuf,
             acc, cnt_stage):
    c = lax.axis_index("c")
    s = lax.axis_index("s")

    _zero_2d(zbuf, SEGS_PER_TILE, D)

    def zc(i, carry):
        cnt2[pl.ds(L * i, L)] = jnp.zeros((L,), jnp.float32)
        return carry

    lax.fori_loop(0, L * B // L, zc, None)

    seg0 = s * SEGS_PER_TILE
    pltpu.sync_copy(zbuf, acc.at[pl.ds(seg0, SEGS_PER_TILE)])
    plsc.subcore_barrier()

    def process(x_hbm, idx_hbm):
        base = s * ROWS_PER_TILE

        def chunk_step(j, carry):
            row0 = base + j * CHUNK
            pltpu.sync_copy(idx_hbm.at[pl.ds(row0, CHUNK)], idxbuf)
            pltpu.sync_copy(x_hbm.at[pl.ds(row0, CHUNK), :], xbuf)
            _count_chunk(cnt2, idxbuf, CHUNK)
            pltpu.sync_copy(xbuf, acc.at[idxbuf], add=True)
            return carry

        @pl.when(s < NUM_SUBCORES - 1)
        def _():
            lax.fori_loop(0, FULL_CHUNKS, chunk_step, None)

        @pl.when(s == NUM_SUBCORES - 1)
        def _():
            lax.fori_loop(0, LAST_FULL_CHUNKS, chunk_step, None)
            row0 = LAST_BASE + LAST_FULL_CHUNKS * CHUNK
            pltpu.sync_copy(idx_hbm.at[pl.ds(row0, TAIL)], idxbuf_t)
            pltpu.sync_copy(x_hbm.at[pl.ds(row0, TAIL), :],
                            xbuf.at[pl.ds(0, TAIL), :])
            _count_chunk(cnt2, idxbuf_t, TAIL)
            pltpu.sync_copy(xbuf.at[pl.ds(0, TAIL), :], acc.at[idxbuf_t],
                            add=True)

    @pl.when(c == 0)
    def _():
        process(xu, bu)

    @pl.when(c == 1)
    def _():
        process(xn, bn)

    def lane_reduce(k, carry):
        tot = jnp.zeros((L,), jnp.float32)
        for r in range(L):
            tot = tot + cnt2[pl.ds(r * B + L * k, L)]
        lcnt[pl.ds(L * k, L)] = tot
        return carry

    lax.fori_loop(0, B // L, lane_reduce, None)
    pltpu.sync_copy(lcnt, cnt_stage.at[s])
    plsc.subcore_barrier()

    pltpu.sync_copy(cnt_stage, cbuf)
    for k in range(SEGS_PER_TILE // L):
        tot = jnp.zeros((L,), jnp.float32)
        for r in range(NUM_SUBCORES):
            tot = tot + cbuf[r, pl.ds(seg0 + L * k, L)]
        csum[pl.ds(L * k, L)] = tot

    pltpu.sync_copy(acc.at[pl.ds(seg0, SEGS_PER_TILE)], dbuf)

    def norm_group(m, carry):
        inv = 1.0 / jnp.maximum(csum[pl.ds(L * m, L)], 1.0)
        for r in range(L):
            k = L * m + r
            scale = jnp.full((L,), inv[r])
            for j in range(D // L):
                dbuf[k, pl.ds(L * j, L)] = dbuf[k, pl.ds(L * j, L)] * scale
        return carry

    lax.fori_loop(0, SEGS_PER_TILE // L, norm_group, None)
    out0 = c * B + seg0
    pltpu.sync_copy(dbuf, means_out.at[pl.ds(out0, SEGS_PER_TILE)])


def _segment_means(x_user, batch_user, x_news, batch_news):
    mesh = plsc.VectorSubcoreMesh(core_axis_name="c", subcore_axis_name="s")
    fn = pl.kernel(
        _sc_body,
        mesh=mesh,
        compiler_params=pltpu.CompilerParams(needs_layout_passes=False),
        out_type=jax.ShapeDtypeStruct((NUM_CORES * B, D), jnp.float32),
        scratch_types=[
            pltpu.VMEM((CHUNK, D), jnp.float32),
            pltpu.VMEM((CHUNK,), jnp.int32),
            pltpu.VMEM((TAIL,), jnp.int32),
            pltpu.VMEM((SEGS_PER_TILE, D), jnp.float32),
            pltpu.VMEM((L * B,), jnp.float32),
            pltpu.VMEM((B,), jnp.float32),
            pltpu.VMEM((NUM_SUBCORES, B), jnp.float32),
            pltpu.VMEM((SEGS_PER_TILE,), jnp.float32),
            pltpu.VMEM((SEGS_PER_TILE, D), jnp.float32),
            pltpu.VMEM_SHARED((B, D), jnp.float32),
            pltpu.VMEM_SHARED((NUM_SUBCORES, B), jnp.float32),
        ],
    )
    return fn(x_user, batch_user, x_news, batch_news)


def _mlp_body(means_ref, ne_ref, w1_ref, b1_ref, w2_ref, b2_ref,
              w3_ref, b3_ref, out_ref):
    pu = means_ref[0:B, :]
    pn = means_ref[B:2 * B, :]
    hp = jax.lax.Precision.HIGHEST
    h = jnp.dot(pu, w1_ref[0:D, :], precision=hp)
    h = h + jnp.dot(pn, w1_ref[D:2 * D, :], precision=hp)
    h = jnp.maximum(h + b1_ref[0:1, :], 0.0)
    h = jnp.maximum(jnp.dot(h, w2_ref[...], precision=hp) + b2_ref[0:1, :], 0.0)
    out_ref[...] = (jnp.dot(h, w3_ref[...], precision=hp) + b3_ref[0:1, :]
                    + ne_ref[...])


def kernel(x_user, batch_user, x_news, batch_news, news_embeddings,
           W1, b1, W2, b2, W3, b3):
    bu = batch_user.astype(jnp.int32)
    bn = batch_news.astype(jnp.int32)
    means = _segment_means(x_user, bu, x_news, bn)
    return pl.pallas_call(
        _mlp_body,
        out_shape=jax.ShapeDtypeStruct((B, D), jnp.float32),
    )(means, news_embeddings,
      W1, b1.reshape(1, D), W2, b2.reshape(1, D), W3, b3.reshape(1, D))


# speedup vs baseline: 8.2776x; 1.5950x over previous
import jax
import jax.numpy as jnp
from jax import lax
from jax.experimental import pallas as pl
from jax.experimental.pallas import tpu as pltpu
from jax.experimental.pallas import tpu_sc as plsc

B = 1024
N = 100000
D = 128

NUM_CORES = 2
NUM_SUBCORES = 16
L = 16
SUB = 128
K = 2
BIG = K * SUB
ROWS_MAIN = 6400
N_BIG_MAIN = ROWS_MAIN // BIG
LAST_BASE = 15 * ROWS_MAIN
N_BIG_LAST = (N - LAST_BASE) // BIG
EXTRA_BASE = LAST_BASE + N_BIG_LAST * BIG
TAIL = N - EXTRA_BASE - SUB
IDX_TROWS = 50
SEGS_PER_TILE = B // NUM_SUBCORES


def _sc_body(xu, bu, xn, bn, means_out,
             xbuf_a, xbuf_b, idx_all, idxbuf_t, cnt2, lcnt, cbuf, csum,
             dbuf, sem_a, sem_b, acc, cnt_stage):
    c = lax.axis_index("c")
    s = lax.axis_index("s")

    def zd(i, carry):
        for j in range(D // L):
            dbuf[i, pl.ds(L * j, L)] = jnp.zeros((L,), jnp.float32)
        return carry

    lax.fori_loop(0, SEGS_PER_TILE, zd, None)

    def zc(i, carry):
        cnt2[pl.ds(L * i, L)] = jnp.zeros((L,), jnp.float32)
        return carry

    lax.fori_loop(0, L * B // L, zc, None)

    seg0 = s * SEGS_PER_TILE
    pltpu.sync_copy(dbuf, acc.at[pl.ds(seg0, SEGS_PER_TILE)])
    plsc.subcore_barrier()

    lane_off = lax.iota(jnp.int32, L) * B
    ones = jnp.ones((L,), jnp.float32)

    def process(x_hbm, idx_hbm, n_big, with_extra):
        base = s * ROWS_MAIN
        pltpu.sync_copy(idx_hbm.at[s], idx_all)

        def start_load(xb, sem, j):
            pltpu.async_copy(x_hbm.at[pl.ds(base + j * BIG, BIG), :], xb, sem)

        def wait_load(xb, sem):
            pltpu.make_async_copy(x_hbm.at[pl.ds(0, BIG), :], xb, sem).wait()

        def counts(row, rows):
            for g in range(rows // L):
                idx16 = idx_all[row, pl.ds(L * g, L)]
                plsc.addupdate_scatter(cnt2, [lane_off + idx16], ones)

        def consume(xb, j):
            for k in range(K):
                row = j * K + k
                counts(row, SUB)
                pltpu.sync_copy(xb.at[pl.ds(SUB * k, SUB), :],
                                acc.at[idx_all.at[row]], add=True)

        start_load(xbuf_a, sem_a, 0)

        def pair(g, carry):
            j0 = 2 * g
            start_load(xbuf_b, sem_b, j0 + 1)
            wait_load(xbuf_a, sem_a)
            consume(xbuf_a, j0)
            start_load(xbuf_a, sem_a, j0 + 2)
            wait_load(xbuf_b, sem_b)
            consume(xbuf_b, j0 + 1)
            return carry

        lax.fori_loop(0, n_big // 2, pair, None)
        wait_load(xbuf_a, sem_a)
        consume(xbuf_a, n_big - 1)

        if with_extra:
            erow = (EXTRA_BASE - LAST_BASE) // SUB
            pltpu.sync_copy(x_hbm.at[pl.ds(EXTRA_BASE, SUB), :],
                            xbuf_a.at[pl.ds(0, SUB), :])
            counts(erow, SUB)
            pltpu.sync_copy(xbuf_a.at[pl.ds(0, SUB), :],
                            acc.at[idx_all.at[erow]], add=True)
            t0 = EXTRA_BASE + SUB
            for g in range(TAIL // L):
                idxbuf_t[pl.ds(L * g, L)] = idx_all[erow + 1, pl.ds(L * g, L)]
            pltpu.sync_copy(x_hbm.at[pl.ds(t0, TAIL), :],
                            xbuf_a.at[pl.ds(0, TAIL), :])
            for g in range(TAIL // L):
                idx16 = idxbuf_t[pl.ds(L * g, L)]
                plsc.addupdate_scatter(cnt2, [lane_off + idx16], ones)
            pltpu.sync_copy(xbuf_a.at[pl.ds(0, TAIL), :],
                            acc.at[idxbuf_t], add=True)

    @pl.when(jnp.logical_and(c == 0, s < NUM_SUBCORES - 1))
    def _():
        process(xu, bu, N_BIG_MAIN, False)

    @pl.when(jnp.logical_and(c == 0, s == NUM_SUBCORES - 1))
    def _():
        process(xu, bu, N_BIG_LAST, True)

    @pl.when(jnp.logical_and(c == 1, s < NUM_SUBCORES - 1))
    def _():
        process(xn, bn, N_BIG_MAIN, False)

    @pl.when(jnp.logical_and(c == 1, s == NUM_SUBCORES - 1))
    def _():
        process(xn, bn, N_BIG_LAST, True)

    def lane_reduce(k, carry):
        tot = jnp.zeros((L,), jnp.float32)
        for r in range(L):
            tot = tot + cnt2[pl.ds(r * B + L * k, L)]
        lcnt[pl.ds(L * k, L)] = tot
        return carry

    lax.fori_loop(0, B // L, lane_reduce, None)
    pltpu.sync_copy(lcnt, cnt_stage.at[s])
    plsc.subcore_barrier()

    pltpu.sync_copy(cnt_stage, cbuf)
    for k in range(SEGS_PER_TILE // L):
        tot = jnp.zeros((L,), jnp.float32)
        for r in range(NUM_SUBCORES):
            tot = tot + cbuf[r, pl.ds(seg0 + L * k, L)]
        csum[pl.ds(L * k, L)] = tot

    pltpu.sync_copy(acc.at[pl.ds(seg0, SEGS_PER_TILE)], dbuf)

    def norm_group(m, carry):
        inv = 1.0 / jnp.maximum(csum[pl.ds(L * m, L)], 1.0)
        for r in range(L):
            k = L * m + r
            scale = jnp.full((L,), inv[r])
            for j in range(D // L):
                dbuf[k, pl.ds(L * j, L)] = dbuf[k, pl.ds(L * j, L)] * scale
        return carry

    lax.fori_loop(0, SEGS_PER_TILE // L, norm_group, None)
    out0 = c * B + seg0
    pltpu.sync_copy(dbuf, means_out.at[pl.ds(out0, SEGS_PER_TILE)])


def _segment_means(x_user, batch_user2d, x_news, batch_news2d):
    mesh = plsc.VectorSubcoreMesh(core_axis_name="c", subcore_axis_name="s")
    fn = pl.kernel(
        _sc_body,
        mesh=mesh,
        compiler_params=pltpu.CompilerParams(needs_layout_passes=False),
        out_type=jax.ShapeDtypeStruct((NUM_CORES * B, D), jnp.float32),
        scratch_types=[
            pltpu.VMEM((BIG, D), jnp.float32),
            pltpu.VMEM((BIG, D), jnp.float32),
            pltpu.VMEM((IDX_TROWS, SUB), jnp.int32),
            pltpu.VMEM((TAIL,), jnp.int32),
            pltpu.VMEM((L * B,), jnp.float32),
            pltpu.VMEM((B,), jnp.float32),
            pltpu.VMEM((NUM_SUBCORES, B), jnp.float32),
            pltpu.VMEM((SEGS_PER_TILE,), jnp.float32),
            pltpu.VMEM((SEGS_PER_TILE, D), jnp.float32),
            pltpu.SemaphoreType.DMA,
            pltpu.SemaphoreType.DMA,
            pltpu.VMEM_SHARED((B, D), jnp.float32),
            pltpu.VMEM_SHARED((NUM_SUBCORES, B), jnp.float32),
        ],
    )
    return fn(x_user, batch_user2d, x_news, batch_news2d)


def _mlp_body(means_ref, ne_ref, w1_ref, b1_ref, w2_ref, b2_ref,
              w3_ref, b3_ref, out_ref):
    pu = means_ref[0:B, :]
    pn = means_ref[B:2 * B, :]
    hp = jax.lax.Precision.HIGHEST
    h = jnp.dot(pu, w1_ref[0:D, :], precision=hp)
    h = h + jnp.dot(pn, w1_ref[D:2 * D, :], precision=hp)
    h = jnp.maximum(h + b1_ref[0:1, :], 0.0)
    h = jnp.maximum(jnp.dot(h, w2_ref[...], precision=hp) + b2_ref[0:1, :], 0.0)
    out_ref[...] = (jnp.dot(h, w3_ref[...], precision=hp) + b3_ref[0:1, :]
                    + ne_ref[...])


def _ids_3d(batch):
    ids = batch.astype(jnp.int32)
    pad = NUM_SUBCORES * IDX_TROWS * SUB - N
    return jnp.pad(ids, (0, pad)).reshape(NUM_SUBCORES, IDX_TROWS, SUB)


def kernel(x_user, batch_user, x_news, batch_news, news_embeddings,
           W1, b1, W2, b2, W3, b3):
    means = _segment_means(x_user, _ids_3d(batch_user),
                           x_news, _ids_3d(batch_news))
    return pl.pallas_call(
        _mlp_body,
        out_shape=jax.ShapeDtypeStruct((B, D), jnp.float32),
    )(means, news_embeddings,
      W1, b1.reshape(1, D), W2, b2.reshape(1, D), W3, b3.reshape(1, D))
